# baseline (speedup 1.0000x reference)
import functools

import jax
import jax.numpy as jnp
from jax import lax
from jax.experimental import pallas as pl
from jax.experimental.pallas import tpu as pltpu
from jax.experimental.pallas import tpu_sc as plsc

_NC = 2
_NS = 16
_NW = _NC * _NS


@functools.lru_cache(maxsize=None)
def _make_gather(BT, S, D, C, NBUF=6):
    B = BT * S
    nch = B // (_NW * C)
    b_per_w = B // _NW
    assert S % b_per_w == 0
    mesh = plsc.VectorSubcoreMesh(core_axis_name="c", subcore_axis_name="s")

    @functools.partial(
        pl.kernel,
        out_type=jax.ShapeDtypeStruct((BT, S, D), jnp.float32),
        mesh=mesh,
        scratch_types=[
            pltpu.VMEM((b_per_w,), jnp.int32),
        ] + [pltpu.VMEM((C, D), jnp.float32) for _ in range(NBUF)]
          + [pltpu.SemaphoreType.DMA for _ in range(2 * NBUF)],
    )
    def k(table_hbm, idx_hbm, out_hbm, idx_v, *rest):
        bufs = rest[:NBUF]
        gsems = rest[NBUF:2 * NBUF]
        ssems = rest[2 * NBUF:]
        wid = lax.axis_index("s") * _NC + lax.axis_index("c")
        base = wid * b_per_w
        bt = base // S
        col = base % S
        pltpu.sync_copy(idx_hbm.at[bt, pl.ds(col, b_per_w)], idx_v)
        LA = 2
        gath = [None] * nch
        scat = [None] * nch
        for j in range(nch + LA):
            if j < nch:
                s = j % NBUF
                if j >= NBUF:
                    scat[j - NBUF].wait()
                gath[j] = pltpu.async_copy(
                    table_hbm.at[idx_v.at[pl.ds(j * C, C)]], bufs[s], gsems[s])
            if j >= LA:
                p = j - LA
                gath[p].wait()
                scat[p] = pltpu.async_copy(
                    bufs[p % NBUF], out_hbm.at[bt, pl.ds(col + p * C, C)],
                    ssems[p % NBUF])
        for p in range(max(0, nch - NBUF), nch):
            scat[p].wait()

    return k


def kernel(x, embedding_weight):
    bt, s = x.shape
    d = embedding_weight.shape[1]
    return _make_gather(bt, s, d, 16)(embedding_weight, x.astype(jnp.int32))
